# baseline (speedup 1.0000x reference)
import functools

import jax
import jax.numpy as jnp
from jax import lax
from jax.experimental import pallas as pl
from jax.experimental.pallas import tpu as pltpu
from jax.experimental.pallas import tpu_sc as plsc

B = 16384
D = 64
H = 128
NC, NS = 2, 16
NW = NC * NS
BPW = B // NW
CH = 64
NCH = BPW // CH
NG = CH // 16


@functools.cache
def _build_sc_gather():
    mesh = plsc.VectorSubcoreMesh(
        core_axis_name="c", subcore_axis_name="s", num_cores=NC, num_subcores=NS
    )

    @functools.partial(
        pl.kernel,
        out_type=(
            jax.ShapeDtypeStruct((B, D), jnp.float32),
            jax.ShapeDtypeStruct((B, D), jnp.float32),
        ),
        mesh=mesh,
        compiler_params=pltpu.CompilerParams(needs_layout_passes=False),
        scratch_types=[
            pltpu.VMEM((NCH, CH), jnp.int32),
            pltpu.VMEM((NCH, CH), jnp.int32),
            pltpu.VMEM((NCH, CH), jnp.int32),
            pltpu.VMEM((NCH, CH), jnp.int32),
            pltpu.VMEM((CH, 2 * D), jnp.float32),
            pltpu.VMEM((CH, 2 * D), jnp.float32),
            pltpu.VMEM((CH, D), jnp.float32),
            pltpu.VMEM((CH, D), jnp.float32),
            pltpu.SemaphoreType.DMA,
        ],
    )
    def sc_gather(ublk_hbm, uoff_hbm, bblk_hbm, boff_hbm, utab_hbm, btab_hbm,
                  uout_hbm, bout_hbm,
                  ublk_v, uoff_v, bblk_v, boff_v, stage_u, stage_b,
                  uout_v, bout_v, sem):
        wid = lax.axis_index("s") * NC + lax.axis_index("c")
        base = wid * BPW
        pltpu.sync_copy(ublk_hbm.at[wid], ublk_v)
        pltpu.sync_copy(uoff_hbm.at[wid], uoff_v)
        pltpu.sync_copy(bblk_hbm.at[wid], bblk_v)
        pltpu.sync_copy(boff_hbm.at[wid], boff_v)

        def chunk_body(j, carry):
            cu = pltpu.async_copy(utab_hbm.at[ublk_v.at[j]], stage_u, sem)
            cb = pltpu.async_copy(btab_hbm.at[bblk_v.at[j]], stage_b, sem)
            cu.wait()
            cb.wait()
            for g in range(NG):
                elpos = lax.iota(jnp.int32, 16) + (g * 16)
                off_u = uoff_v[j, pl.ds(g * 16, 16)]
                off_b = boff_v[j, pl.ds(g * 16, 16)]
                for d in range(D):
                    dvec = jnp.full((16,), d, jnp.int32)
                    uv = plsc.load_gather(stage_u, [elpos, off_u + dvec])
                    plsc.store_scatter(uout_v, [elpos, dvec], uv)
                    bv = plsc.load_gather(stage_b, [elpos, off_b + dvec])
                    plsc.store_scatter(bout_v, [elpos, dvec], bv)
            pltpu.sync_copy(uout_v, uout_hbm.at[pl.ds(base + j * CH, CH)])
            pltpu.sync_copy(bout_v, bout_hbm.at[pl.ds(base + j * CH, CH)])
            return carry

        lax.fori_loop(0, NCH, chunk_body, 0)

    return sc_gather


BLK = 2048


def _tc_body(tag_ref, u_ref, bk_ref, w_ref, b_ref, out_ref):
    proj = jnp.dot(tag_ref[...], w_ref[...],
                   preferred_element_type=jnp.float32) + b_ref[...]
    out_ref[...] = jnp.sum(u_ref[...] * (bk_ref[...] + proj), axis=1)


def _tc_combine(tag, u_emb, bk_emb, w_lin, b2d):
    return pl.pallas_call(
        _tc_body,
        grid=(B // BLK,),
        in_specs=[
            pl.BlockSpec((BLK, H), lambda i: (i, 0)),
            pl.BlockSpec((BLK, D), lambda i: (i, 0)),
            pl.BlockSpec((BLK, D), lambda i: (i, 0)),
            pl.BlockSpec((H, D), lambda i: (0, 0)),
            pl.BlockSpec((1, D), lambda i: (0, 0)),
        ],
        out_specs=pl.BlockSpec((BLK,), lambda i: (i,)),
        out_shape=jax.ShapeDtypeStruct((B,), jnp.float32),
    )(tag, u_emb, bk_emb, w_lin, b2d)


def kernel(user, book, tag_embedding, user_table, book_table, W_lin, b_lin):
    ublk = (user >> 1).reshape(NW, NCH, CH)
    uoff = ((user & 1) << 6).reshape(NW, NCH, CH)
    bblk = (book >> 1).reshape(NW, NCH, CH)
    boff = ((book & 1) << 6).reshape(NW, NCH, CH)
    utab2 = user_table.reshape(-1, 2 * D)
    btab2 = book_table.reshape(-1, 2 * D)
    u_emb, bk_emb = _build_sc_gather()(ublk, uoff, bblk, boff, utab2, btab2)
    return _tc_combine(tag_embedding, u_emb, bk_emb, W_lin,
                       b_lin.reshape(1, D))


# speedup vs baseline: 1.8159x; 1.8159x over previous
import functools

import jax
import jax.numpy as jnp
from jax import lax
from jax.experimental import pallas as pl
from jax.experimental.pallas import tpu as pltpu
from jax.experimental.pallas import tpu_sc as plsc

B = 16384
D = 64
H = 128
NC, NS = 2, 16
NW = NC * NS
BPW = B // NW
CH = 64
NCH = BPW // CH


@functools.cache
def _build_sc_gather():
    mesh = plsc.VectorSubcoreMesh(
        core_axis_name="c", subcore_axis_name="s", num_cores=NC, num_subcores=NS
    )

    @functools.partial(
        pl.kernel,
        out_type=(
            jax.ShapeDtypeStruct((B, D), jnp.float32),
            jax.ShapeDtypeStruct((B, D), jnp.float32),
        ),
        mesh=mesh,
        compiler_params=pltpu.CompilerParams(needs_layout_passes=False),
        scratch_types=[
            pltpu.VMEM((BPW,), jnp.int32),
            pltpu.VMEM((BPW,), jnp.int32),
            pltpu.VMEM((CH, D), jnp.float32),
            pltpu.VMEM((CH, D), jnp.float32),
            pltpu.SemaphoreType.DMA,
            pltpu.SemaphoreType.DMA,
        ],
    )
    def sc_gather(uidx_hbm, bidx_hbm, utab_hbm, btab_hbm,
                  uout_hbm, bout_hbm,
                  uidx_v, bidx_v, stage_u, stage_b, semu, semb):
        wid = lax.axis_index("s") * NC + lax.axis_index("c")
        base = wid * BPW
        pltpu.sync_copy(uidx_hbm.at[pl.ds(base, BPW)], uidx_v)
        pltpu.sync_copy(bidx_hbm.at[pl.ds(base, BPW)], bidx_v)

        def chunk_body(j, carry):
            for g in range(CH // 16):
                uvec = uidx_v[pl.ds(j * CH + g * 16, 16)]
                bvec = bidx_v[pl.ds(j * CH + g * 16, 16)]
                for l in range(16):
                    e = g * 16 + l
                    pltpu.make_async_copy(
                        utab_hbm.at[pl.ds(uvec[l], 1)],
                        stage_u.at[pl.ds(e, 1)], semu,
                    ).start()
                    pltpu.make_async_copy(
                        btab_hbm.at[pl.ds(bvec[l], 1)],
                        stage_b.at[pl.ds(e, 1)], semb,
                    ).start()
            pltpu.make_async_copy(utab_hbm.at[pl.ds(0, CH)], stage_u, semu).wait()
            pltpu.make_async_copy(btab_hbm.at[pl.ds(0, CH)], stage_b, semb).wait()
            pltpu.sync_copy(stage_u, uout_hbm.at[pl.ds(base + j * CH, CH)])
            pltpu.sync_copy(stage_b, bout_hbm.at[pl.ds(base + j * CH, CH)])
            return carry

        lax.fori_loop(0, NCH, chunk_body, 0)

    return sc_gather


BLK = 2048


def _tc_body(tag_ref, u_ref, bk_ref, w_ref, b_ref, out_ref):
    proj = jnp.dot(tag_ref[...], w_ref[...],
                   preferred_element_type=jnp.float32) + b_ref[...]
    out_ref[...] = jnp.sum(u_ref[...] * (bk_ref[...] + proj), axis=1)


def _tc_combine(tag, u_emb, bk_emb, w_lin, b2d):
    return pl.pallas_call(
        _tc_body,
        grid=(B // BLK,),
        in_specs=[
            pl.BlockSpec((BLK, H), lambda i: (i, 0)),
            pl.BlockSpec((BLK, D), lambda i: (i, 0)),
            pl.BlockSpec((BLK, D), lambda i: (i, 0)),
            pl.BlockSpec((H, D), lambda i: (0, 0)),
            pl.BlockSpec((1, D), lambda i: (0, 0)),
        ],
        out_specs=pl.BlockSpec((BLK,), lambda i: (i,)),
        out_shape=jax.ShapeDtypeStruct((B,), jnp.float32),
    )(tag, u_emb, bk_emb, w_lin, b2d)


def kernel(user, book, tag_embedding, user_table, book_table, W_lin, b_lin):
    u_emb, bk_emb = _build_sc_gather()(user, book, user_table, book_table)
    return _tc_combine(tag_embedding, u_emb, bk_emb, W_lin,
                       b_lin.reshape(1, D))
